# baseline (speedup 1.0000x reference)
import functools

import jax
import jax.numpy as jnp
from jax import lax
from jax.experimental import pallas as pl
from jax.experimental.pallas import tpu as pltpu
from jax.experimental.pallas import tpu_sc as plsc

NC, NS = 2, 16
NW = NC * NS
ROW = 128
K = 4
HID = 64


@functools.lru_cache(maxsize=None)
def _make_transpose(vocab: int, hid: int):
    VB = 1280
    grid = (vocab + VB - 1) // VB

    def body(in_ref, out_ref):
        out_ref[:, 0:hid] = jnp.transpose(in_ref[...], (1, 0))

    return pl.pallas_call(
        body,
        grid=(grid,),
        in_specs=[pl.BlockSpec((hid, VB), lambda i: (0, i))],
        out_specs=pl.BlockSpec((VB, 128), lambda i: (i, 0)),
        out_shape=jax.ShapeDtypeStruct((vocab, 128), jnp.float32),
    )


@functools.lru_cache(maxsize=None)
def _make_gather(n_rows: int):
    rows_per_w = n_rows // NW
    groups = rows_per_w // K
    mesh = plsc.VectorSubcoreMesh(core_axis_name="c", subcore_axis_name="s")

    @functools.partial(
        pl.kernel,
        out_type=jax.ShapeDtypeStruct((n_rows, ROW, HID), jnp.float32),
        mesh=mesh,
        scratch_types=[
            pltpu.VMEM((K, ROW), jnp.int32),
            pltpu.VMEM((K, ROW, 2 * HID), jnp.float32),
            pltpu.SemaphoreType.DMA,
        ],
        compiler_params=pltpu.CompilerParams(use_tc_tiling_on_sc=False),
    )
    def k(ids_hbm, table_hbm, out_hbm, idx_v, rows_v, sem):
        wid = lax.axis_index("s") * NC + lax.axis_index("c")
        row_base = wid * rows_per_w

        @pl.loop(0, groups)
        def _group(g):
            r0 = row_base + g * K
            pltpu.sync_copy(ids_hbm.at[pl.ds(r0, K)], idx_v)
            cps = [
                pltpu.async_copy(table_hbm.at[idx_v.at[j]], rows_v.at[j], sem)
                for j in range(K)
            ]
            for cp in cps:
                cp.wait()
            pltpu.sync_copy(rows_v.at[:, :, pl.ds(0, HID)],
                            out_hbm.at[pl.ds(r0, K)])

    return k


def kernel(input_ids, wte):
    B, L = input_ids.shape
    V, H = wte.shape
    n = B * L
    n_rows = n // ROW
    ids = input_ids.reshape(n_rows, ROW).astype(jnp.int32)
    table = _make_transpose(V, H)(wte.T)
    out = _make_gather(n_rows)(ids, table)
    return out.reshape(B, L, HID)
